# baseline (speedup 1.0000x reference)
import functools

import jax
import jax.numpy as jnp
from jax import lax
from jax.experimental import pallas as pl
from jax.experimental.pallas import tpu as pltpu
from jax.experimental.pallas import tpu_sc as plsc

_D = 64
_NFS = 16
_NW = 32
_CHUNK = 128


@functools.lru_cache(maxsize=None)
def _build(B, n_rows):
    n_per_w = B // _NW
    n_groups = n_per_w // 16
    n_chunks = n_per_w // _CHUNK

    mesh = plsc.VectorSubcoreMesh(core_axis_name="c", subcore_axis_name="s")

    @functools.partial(
        pl.kernel,
        mesh=mesh,
        out_type=jax.ShapeDtypeStruct((B,), jnp.float32),
        compiler_params=pltpu.CompilerParams(
            needs_layout_passes=False, use_tc_tiling_on_sc=True),
        scratch_types=[
            pltpu.VMEM((n_per_w, _D), jnp.float32),
            pltpu.VMEM((_CHUNK, 2 * _D), jnp.float32),
            pltpu.VMEM((n_per_w,), jnp.float32),
            pltpu.VMEM((n_per_w,), jnp.int32),
            pltpu.VMEM((n_per_w,), jnp.int32),
            pltpu.VMEM((n_per_w,), jnp.int32),
            [pltpu.VMEM((_CHUNK,), jnp.int32)] * 4,
            [pltpu.VMEM((_CHUNK,), jnp.int32)] * 4,
            pltpu.VMEM((16, 16), jnp.float32),
            pltpu.VMEM((16, 16), jnp.float32),
            pltpu.VMEM((n_per_w,), jnp.float32),
            pltpu.SemaphoreType.DMA,
            pltpu.SemaphoreType.DMA,
            pltpu.SemaphoreType.DMA,
            pltpu.SemaphoreType.DMA,
        ],
    )
    def k(feat_hbm, stn_hbm, fct_hbm, stp_hbm, wt_hbm, bias_hbm, out_hbm,
          feat_v, w_v, bias_v, stn_v, fct_v, stp_v, idx_c, idxw_c,
          scr_lo, scr_hi, out_v, sem_f, sem_id, sem_b, sem_w):
        wid = lax.axis_index("s") * 2 + lax.axis_index("c")
        base = wid * n_per_w

        cp_f = pltpu.async_copy(feat_hbm.at[pl.ds(base, n_per_w)], feat_v, sem_f)
        cp_s = pltpu.async_copy(stn_hbm.at[pl.ds(base, n_per_w)], stn_v, sem_id)
        cp_c = pltpu.async_copy(fct_hbm.at[pl.ds(base, n_per_w)], fct_v, sem_id)
        cp_p = pltpu.async_copy(stp_hbm.at[pl.ds(base, n_per_w)], stp_v, sem_id)
        cp_s.wait()
        cp_c.wait()
        cp_p.wait()

        for c in range(n_per_w // 16):
            o = c * 16
            st = stn_v[pl.ds(o, 16)]
            fo = fct_v[pl.ds(o, 16)]
            sp = stp_v[pl.ds(o, 16)]
            flat = st * _NFS + (fo & 1) * 8 + (sp & 7)
            kk = o // _CHUNK
            idx_c[kk][pl.ds(o - kk * _CHUNK, 16)] = flat
            idxw_c[kk][pl.ds(o - kk * _CHUNK, 16)] = flat >> 1
            stp_v[pl.ds(o, 16)] = flat & 1

        bias_descs = []
        for kk in range(n_chunks):
            bias_descs.append(pltpu.async_copy(
                bias_hbm.at[idx_c[kk]],
                bias_v.at[pl.ds(kk * _CHUNK, _CHUNK)], sem_b))
        cp_f.wait()

        iota16 = lax.iota(jnp.int32, 16)

        for kk in range(n_chunks):
            cp_w = pltpu.async_copy(wt_hbm.at[idxw_c[kk]], w_v, sem_w)
            if kk == 0:
                for d in bias_descs:
                    d.wait()
            cp_w.wait()

            def group(gl, carry):
                e0 = kk * _CHUNK + gl * 16
                for e in range(16):
                    row = e0 + e
                    rloc = gl * 16 + e
                    f0 = feat_v[row, pl.ds(0, 16)]
                    acc_lo = w_v[rloc, pl.ds(0, 16)] * f0
                    acc_hi = w_v[rloc, pl.ds(_D, 16)] * f0
                    for t in range(1, _D // 16):
                        ft = feat_v[row, pl.ds(t * 16, 16)]
                        acc_lo = acc_lo + w_v[rloc, pl.ds(t * 16, 16)] * ft
                        acc_hi = acc_hi + w_v[rloc, pl.ds(_D + t * 16, 16)] * ft
                    scr_lo[e, pl.ds(0, 16)] = acc_lo
                    scr_hi[e, pl.ds(0, 16)] = acc_hi
                lo = scr_lo[0, pl.ds(0, 16)] * 0.0
                hi = lo
                for l in range(16):
                    lvec = jnp.full((16,), l, jnp.int32)
                    lo = lo + plsc.load_gather(scr_lo, [iota16, lvec])
                    hi = hi + plsc.load_gather(scr_hi, [iota16, lvec])
                par = stp_v[pl.ds(e0, 16)].astype(jnp.float32)
                tot = bias_v[pl.ds(e0, 16)] + lo + par * (hi - lo)
                out_v[pl.ds(e0, 16)] = tot
                return carry

            lax.fori_loop(0, _CHUNK // 16, group, 0)
        pltpu.sync_copy(out_v, out_hbm.at[pl.ds(base, n_per_w)])

    return k


def kernel(features, station_id, forecast_id, step_id, weights, biases):
    B = features.shape[0]
    n_stations = weights.shape[0]
    n_rows = n_stations * _NFS
    k = _build(B, n_rows)
    return k(features,
             station_id.astype(jnp.int32),
             forecast_id.astype(jnp.int32),
             step_id.astype(jnp.int32),
             weights.reshape(n_rows // 2, 2 * _D),
             biases.reshape(n_rows))


# speedup vs baseline: 1.0113x; 1.0113x over previous
import functools

import jax
import jax.numpy as jnp
from jax import lax
from jax.experimental import pallas as pl
from jax.experimental.pallas import tpu as pltpu
from jax.experimental.pallas import tpu_sc as plsc

_D = 64
_NFS = 16
_NW = 32
_CHUNK = 128


@functools.lru_cache(maxsize=None)
def _build(B, n_rows):
    n_per_w = B // _NW
    n_groups = n_per_w // 16
    n_chunks = n_per_w // _CHUNK

    mesh = plsc.VectorSubcoreMesh(core_axis_name="c", subcore_axis_name="s")

    @functools.partial(
        pl.kernel,
        mesh=mesh,
        out_type=jax.ShapeDtypeStruct((B,), jnp.float32),
        compiler_params=pltpu.CompilerParams(
            needs_layout_passes=False, use_tc_tiling_on_sc=False),
        scratch_types=[
            pltpu.VMEM((n_per_w, _D), jnp.float32),
            pltpu.VMEM((n_per_w, _D), jnp.float32),
            pltpu.VMEM((n_per_w,), jnp.float32),
            pltpu.VMEM((n_per_w,), jnp.int32),
            pltpu.VMEM((n_per_w,), jnp.int32),
            pltpu.VMEM((n_per_w,), jnp.int32),
            [pltpu.VMEM((_CHUNK,), jnp.int32)] * 4,
            pltpu.VMEM((16, 16), jnp.float32),
            pltpu.VMEM((n_per_w,), jnp.float32),
            pltpu.SemaphoreType.DMA,
            pltpu.SemaphoreType.DMA,
            pltpu.SemaphoreType.DMA,
            pltpu.SemaphoreType.DMA,
        ],
    )
    def k(feat_hbm, stn_hbm, fct_hbm, stp_hbm, wt_hbm, bias_hbm, out_hbm,
          feat_v, w_v, bias_v, stn_v, fct_v, stp_v, idx_c, scr,
          out_v, sem_f, sem_id, sem_b, sem_w):
        wid = lax.axis_index("s") * 2 + lax.axis_index("c")
        base = wid * n_per_w

        cp_f = pltpu.async_copy(feat_hbm.at[pl.ds(base, n_per_w)], feat_v, sem_f)
        cp_s = pltpu.async_copy(stn_hbm.at[pl.ds(base, n_per_w)], stn_v, sem_id)
        cp_c = pltpu.async_copy(fct_hbm.at[pl.ds(base, n_per_w)], fct_v, sem_id)
        cp_p = pltpu.async_copy(stp_hbm.at[pl.ds(base, n_per_w)], stp_v, sem_id)
        cp_s.wait()
        cp_c.wait()
        cp_p.wait()

        for c in range(n_per_w // 16):
            o = c * 16
            st = stn_v[pl.ds(o, 16)]
            fo = fct_v[pl.ds(o, 16)]
            sp = stp_v[pl.ds(o, 16)]
            flat = st * _NFS + (fo & 1) * 8 + (sp & 7)
            kk = o // _CHUNK
            idx_c[kk][pl.ds(o - kk * _CHUNK, 16)] = flat

        gather_descs = []
        for kk in range(n_chunks):
            gather_descs.append(pltpu.async_copy(
                wt_hbm.at[idx_c[kk]],
                w_v.at[pl.ds(kk * _CHUNK, _CHUNK)], sem_w))
            gather_descs.append(pltpu.async_copy(
                bias_hbm.at[idx_c[kk]],
                bias_v.at[pl.ds(kk * _CHUNK, _CHUNK)], sem_b))
        cp_f.wait()
        for d in gather_descs:
            d.wait()

        iota16 = lax.iota(jnp.int32, 16)

        def group(g, carry):
            e0 = g * 16
            for e in range(16):
                row = e0 + e
                acc = w_v[row, pl.ds(0, 16)] * feat_v[row, pl.ds(0, 16)]
                for t in range(1, _D // 16):
                    acc = acc + (w_v[row, pl.ds(t * 16, 16)]
                                 * feat_v[row, pl.ds(t * 16, 16)])
                scr[e, pl.ds(0, 16)] = acc
            tot = bias_v[pl.ds(e0, 16)]
            for l in range(16):
                lvec = jnp.full((16,), l, jnp.int32)
                tot = tot + plsc.load_gather(scr, [iota16, lvec])
            out_v[pl.ds(e0, 16)] = tot
            return carry

        lax.fori_loop(0, n_groups, group, 0)
        pltpu.sync_copy(out_v, out_hbm.at[pl.ds(base, n_per_w)])

    return k


def kernel(features, station_id, forecast_id, step_id, weights, biases):
    B = features.shape[0]
    n_stations = weights.shape[0]
    n_rows = n_stations * _NFS
    k = _build(B, n_rows)
    return k(features,
             station_id.astype(jnp.int32),
             forecast_id.astype(jnp.int32),
             step_id.astype(jnp.int32),
             weights.reshape(n_rows, _D),
             biases.reshape(n_rows))
